# Initial kernel scaffold; baseline (speedup 1.0000x reference)
import jax
import jax.numpy as jnp
from jax.experimental import pallas as pl


def kernel(x, edge_index, batch, W0, b0, W1, b1, W2, b2, fW1, fb1, fW2, fb2):
    raise NotImplementedError("write your pallas kernel here")


# speedup vs baseline: 13.0348x; 13.0348x over previous
import functools

import jax
import jax.numpy as jnp
from jax import lax
from jax.experimental import pallas as pl
from jax.experimental.pallas import tpu as pltpu
from jax.experimental.pallas import tpu_sc as plsc

N = 100000
E = 1600000
H = 32
HH = 16
G = 64
DENSE = 64

NC = 2
NS = 16

EPT = 100352
E_PAD = EPT * NS
MAC = 8
NMAC = EPT // (MAC * 128)
EPT_D = E_PAD // 2 // NS
MAC_D = 8
NMAC_D = EPT_D // (MAC_D * 128)

N_ACC = 100352
N_ACC1 = 100096
RPT = N_ACC // NS

_mesh = plsc.VectorSubcoreMesh(core_axis_name="c", subcore_axis_name="s")


def _deg_body(srcdst_hbm, out_hbm, idx_d, ones_v, zbuf, acc):
    c = lax.axis_index("c")
    s = lax.axis_index("s")

    def init(i, _):
        zbuf[pl.ds(i * 16, 16)] = jnp.zeros((16,), jnp.float32)
        return 0
    lax.fori_loop(0, N_ACC1 // NS // 16, init, 0)
    for j in range(8):
        ones_v[pl.ds(j * 16, 16)] = jnp.ones((16,), jnp.float32)
    pltpu.sync_copy(
        zbuf, acc.at[pl.ds(pl.multiple_of(s * (N_ACC1 // NS), 8),
                           N_ACC1 // NS)])
    plsc.subcore_barrier()

    base_row = (c * (E_PAD // 2) + s * EPT_D) // 128

    def body(i, _):
        r = pl.multiple_of(base_row + i * MAC_D, 8)
        pltpu.sync_copy(srcdst_hbm.at[1, pl.ds(r, MAC_D)], idx_d)
        for j in range(MAC_D):
            pltpu.sync_copy(ones_v, acc.at[idx_d.at[j]], add=True)
        return 0
    lax.fori_loop(0, NMAC_D, body, 0)

    plsc.subcore_barrier()
    r0 = pl.multiple_of(s * (N_ACC1 // NS), 8)
    ro = pl.multiple_of(c * N_ACC1 + s * (N_ACC1 // NS), 8)
    pltpu.sync_copy(acc.at[pl.ds(r0, N_ACC1 // NS)], zbuf)
    pltpu.sync_copy(zbuf, out_hbm.at[pl.ds(ro, N_ACC1 // NS)])


_sc_params = pltpu.CompilerParams(use_tc_tiling_on_sc=False)

_deg_kernel = pl.kernel(
    _deg_body,
    out_type=jax.ShapeDtypeStruct((NC * N_ACC1,), jnp.float32),
    mesh=_mesh,
    compiler_params=_sc_params,
    scratch_types=[
        pltpu.VMEM((MAC_D, 128), jnp.int32),
        pltpu.VMEM((128,), jnp.float32),
        pltpu.VMEM((N_ACC1 // NS,), jnp.float32),
        pltpu.VMEM_SHARED((N_ACC1,), jnp.float32),
    ],
)


RB = MAC * 128


def _scatter_body(zs_a, zs_b, srcdst_hbm, out_hbm,
                  idx_s, idx_d, rows, acc, sem):
    c = lax.axis_index("c")
    s = lax.axis_index("s")

    def init(i, _):
        rows[i, :] = jnp.zeros((16,), jnp.float32)
        return 0
    lax.fori_loop(0, RB, init, 0)
    for j in range(RPT // RB):
        pltpu.sync_copy(
            rows, acc.at[pl.ds(pl.multiple_of(s * RPT + j * RB, 8), RB)])
    pltpu.sync_copy(
        rows.at[pl.ds(0, RPT % RB)],
        acc.at[pl.ds(pl.multiple_of(s * RPT + RB * (RPT // RB), 8),
                     RPT % RB)])
    plsc.subcore_barrier()

    base_row = s * EPT // 128

    def run(table):
        def body(i, _):
            r = pl.multiple_of(base_row + i * MAC, 8)
            pltpu.sync_copy(srcdst_hbm.at[0, pl.ds(r, MAC)], idx_s)
            pltpu.sync_copy(srcdst_hbm.at[1, pl.ds(r, MAC)], idx_d)
            for j in range(MAC):
                pltpu.async_copy(table.at[idx_s.at[j]],
                                 rows.at[pl.ds(j * 128, 128)], sem).wait()
                pltpu.sync_copy(rows.at[pl.ds(j * 128, 128)],
                                acc.at[idx_d.at[j]], add=True)
            return 0
        lax.fori_loop(0, NMAC, body, 0)

    @pl.when(c == 0)
    def _():
        run(zs_a)

    @pl.when(c == 1)
    def _():
        run(zs_b)

    plsc.subcore_barrier()
    for j in range(RPT // RB):
        r0 = pl.multiple_of(s * RPT + j * RB, 8)
        pltpu.sync_copy(acc.at[pl.ds(r0, RB)], rows)
        pltpu.sync_copy(rows, out_hbm.at[c, pl.ds(r0, RB)])
    rt = pl.multiple_of(s * RPT + RB * (RPT // RB), 8)
    pltpu.sync_copy(acc.at[pl.ds(rt, RPT % RB)],
                    rows.at[pl.ds(0, RPT % RB)])
    pltpu.sync_copy(rows.at[pl.ds(0, RPT % RB)],
                    out_hbm.at[c, pl.ds(rt, RPT % RB)])


_scatter_kernel = pl.kernel(
    _scatter_body,
    out_type=jax.ShapeDtypeStruct((NC, N_ACC, HH), jnp.float32),
    mesh=_mesh,
    compiler_params=_sc_params,
    scratch_types=[
        pltpu.VMEM((MAC, 128), jnp.int32),
        pltpu.VMEM((MAC, 128), jnp.int32),
        pltpu.VMEM((RB, HH), jnp.float32),
        pltpu.VMEM_SHARED((N_ACC, HH), jnp.float32),
        pltpu.SemaphoreType.DMA,
    ],
)


BN = 1000
NBLK = N // BN


def _k0_body(p0, p1, x, w0, dinv_o, z_o, zsa_o, zsb_o):
    deg = p0[...] + p1[...] + 1.0
    dinv = lax.rsqrt(deg)
    z = jnp.dot(x[...], w0[...], preferred_element_type=jnp.float32)
    zs = z * dinv
    dinv_o[...] = dinv
    z_o[...] = z
    zsa_o[...] = zs[:, :HH]
    zsb_o[...] = zs[:, HH:]


def _k0(p0, p1, xpad, w0pad):
    return pl.pallas_call(
        _k0_body,
        grid=(NBLK,),
        in_specs=[
            pl.BlockSpec((BN, 1), lambda i: (i, 0)),
            pl.BlockSpec((BN, 1), lambda i: (i, 0)),
            pl.BlockSpec((BN, 8), lambda i: (i, 0)),
            pl.BlockSpec((8, H), lambda i: (0, 0)),
        ],
        out_specs=[
            pl.BlockSpec((BN, 1), lambda i: (i, 0)),
            pl.BlockSpec((BN, H), lambda i: (i, 0)),
            pl.BlockSpec((BN, HH), lambda i: (i, 0)),
            pl.BlockSpec((BN, HH), lambda i: (i, 0)),
        ],
        out_shape=[
            jax.ShapeDtypeStruct((N, 1), jnp.float32),
            jax.ShapeDtypeStruct((N, H), jnp.float32),
            jax.ShapeDtypeStruct((N, HH), jnp.float32),
            jax.ShapeDtypeStruct((N, HH), jnp.float32),
        ],
    )(p0, p1, xpad, w0pad)


def _kmid_body(sa, sb, z_prev, dinv, b, w, z_o, zsa_o, zsb_o):
    di = dinv[...]
    agg = jnp.concatenate([sa[...], sb[...]], axis=1) * di \
        + z_prev[...] * (di * di)
    h = jnp.maximum(agg + b[...], 0.0)
    z = jnp.dot(h, w[...], preferred_element_type=jnp.float32)
    zs = z * di
    z_o[...] = z
    zsa_o[...] = zs[:, :HH]
    zsb_o[...] = zs[:, HH:]


def _kmid(sa, sb, z_prev, dinv, b, w):
    return pl.pallas_call(
        _kmid_body,
        grid=(NBLK,),
        in_specs=[
            pl.BlockSpec((BN, HH), lambda i: (i, 0)),
            pl.BlockSpec((BN, HH), lambda i: (i, 0)),
            pl.BlockSpec((BN, H), lambda i: (i, 0)),
            pl.BlockSpec((BN, 1), lambda i: (i, 0)),
            pl.BlockSpec((1, H), lambda i: (0, 0)),
            pl.BlockSpec((H, H), lambda i: (0, 0)),
        ],
        out_specs=[
            pl.BlockSpec((BN, H), lambda i: (i, 0)),
            pl.BlockSpec((BN, HH), lambda i: (i, 0)),
            pl.BlockSpec((BN, HH), lambda i: (i, 0)),
        ],
        out_shape=[
            jax.ShapeDtypeStruct((N, H), jnp.float32),
            jax.ShapeDtypeStruct((N, HH), jnp.float32),
            jax.ShapeDtypeStruct((N, HH), jnp.float32),
        ],
    )(sa, sb, z_prev, dinv, b, w)


def _kfin_body(sa, sb, z_prev, dinv, b, batch, fw1, fb1, fw2, fb2,
               out_o, sums, cnt):
    i = pl.program_id(0)

    @pl.when(i == 0)
    def _():
        sums[...] = jnp.zeros_like(sums)
        cnt[...] = jnp.zeros_like(cnt)

    di = dinv[...]
    agg = jnp.concatenate([sa[...], sb[...]], axis=1) * di \
        + z_prev[...] * (di * di)
    h = jnp.maximum(agg + b[...], 0.0)

    bb = batch[...].reshape(1, BN)
    gid = lax.broadcasted_iota(jnp.int32, (G, BN), 0)
    onehot = (gid == bb).astype(jnp.float32)
    sums[...] += jnp.dot(onehot, h, preferred_element_type=jnp.float32)
    cnt[...] += jnp.sum(onehot, axis=1, keepdims=True)

    @pl.when(i == NBLK - 1)
    def _():
        pooled = sums[...] / jnp.maximum(cnt[...], 1.0)
        r = jnp.maximum(
            jnp.dot(pooled, fw1[...], preferred_element_type=jnp.float32)
            + fb1[...], 0.0)
        out_o[...] = (
            jnp.dot(r, fw2[...], preferred_element_type=jnp.float32)
            + fb2[...])


def _kfin(sa, sb, z_prev, dinv, b, batch3, fw1, fb1, fw2, fb2):
    return pl.pallas_call(
        _kfin_body,
        grid=(NBLK,),
        in_specs=[
            pl.BlockSpec((BN, HH), lambda i: (i, 0)),
            pl.BlockSpec((BN, HH), lambda i: (i, 0)),
            pl.BlockSpec((BN, H), lambda i: (i, 0)),
            pl.BlockSpec((BN, 1), lambda i: (i, 0)),
            pl.BlockSpec((1, H), lambda i: (0, 0)),
            pl.BlockSpec((1, 1, BN), lambda i: (i, 0, 0)),
            pl.BlockSpec((H, DENSE), lambda i: (0, 0)),
            pl.BlockSpec((1, DENSE), lambda i: (0, 0)),
            pl.BlockSpec((DENSE, 1), lambda i: (0, 0)),
            pl.BlockSpec((1, 1), lambda i: (0, 0)),
        ],
        out_specs=pl.BlockSpec((G, 1), lambda i: (0, 0)),
        out_shape=jax.ShapeDtypeStruct((G, 1), jnp.float32),
        scratch_shapes=[
            pltpu.VMEM((G, H), jnp.float32),
            pltpu.VMEM((G, 1), jnp.float32),
        ],
    )(sa, sb, z_prev, dinv, b, batch3, fw1, fb1, fw2, fb2)


@jax.jit
def kernel(x, edge_index, batch, W0, b0, W1, b1, W2, b2, fW1, fb1, fW2, fb2):
    pad = E_PAD - E
    srcp = jnp.concatenate([edge_index[0], jnp.zeros((pad,), jnp.int32)])
    dstp = jnp.concatenate([edge_index[1], jnp.full((pad,), N, jnp.int32)])
    srcdst = jnp.stack([srcp.reshape(E_PAD // 128, 128),
                        dstp.reshape(E_PAD // 128, 128)])

    xpad = jnp.pad(x, ((0, 0), (0, 8 - x.shape[1])))
    w0pad = jnp.pad(W0, ((0, 8 - W0.shape[0]), (0, 0)))

    degp = _deg_kernel(srcdst)
    p0 = degp[:N].reshape(N, 1)
    p1 = degp[N_ACC1:N_ACC1 + N].reshape(N, 1)

    dinv, z0, zs0a, zs0b = _k0(p0, p1, xpad, w0pad)
    s0 = _scatter_kernel(zs0a, zs0b, srcdst)
    z1, zs1a, zs1b = _kmid(s0[0, :N], s0[1, :N], z0, dinv,
                           b0.reshape(1, H), W1)
    s1 = _scatter_kernel(zs1a, zs1b, srcdst)
    z2, zs2a, zs2b = _kmid(s1[0, :N], s1[1, :N], z1, dinv,
                           b1.reshape(1, H), W2)
    s2 = _scatter_kernel(zs2a, zs2b, srcdst)

    batch3 = batch.reshape(NBLK, 1, BN)
    return _kfin(s2[0, :N], s2[1, :N], z2, dinv, b2.reshape(1, H), batch3,
                 fW1, fb1.reshape(1, DENSE), fW2, fb2.reshape(1, 1))


# speedup vs baseline: 20.7814x; 1.5943x over previous
import functools

import jax
import jax.numpy as jnp
from jax import lax
from jax.experimental import pallas as pl
from jax.experimental.pallas import tpu as pltpu
from jax.experimental.pallas import tpu_sc as plsc

N = 100000
E = 1600000
H = 32
HH = 16
G = 64
DENSE = 64

NC = 2
NS = 16

EPT = 100352
E_PAD = EPT * NS
MAC = 4
MACE = MAC * 128
NMAC = EPT // MACE
NBUF = 3
EPT_D = E_PAD // 2 // NS
MAC_D = 8
NMAC_D = EPT_D // (MAC_D * 128)

N_ACC = 100352
N_ACC1 = 100096
RPT = N_ACC // NS

_mesh = plsc.VectorSubcoreMesh(core_axis_name="c", subcore_axis_name="s")


def _deg_body(dst2_hbm, out_hbm, idx_d, ones_v, zbuf, acc):
    c = lax.axis_index("c")
    s = lax.axis_index("s")

    def init(i, _):
        zbuf[pl.ds(i * 16, 16)] = jnp.zeros((16,), jnp.float32)
        return 0
    lax.fori_loop(0, N_ACC1 // NS // 16, init, 0)
    for j in range(8):
        ones_v[pl.ds(j * 16, 16)] = jnp.ones((16,), jnp.float32)
    pltpu.sync_copy(
        zbuf, acc.at[pl.ds(pl.multiple_of(s * (N_ACC1 // NS), 8),
                           N_ACC1 // NS)])
    plsc.subcore_barrier()

    base_row = (c * (E_PAD // 2) + s * EPT_D) // 128

    def body(i, _):
        r = pl.multiple_of(base_row + i * MAC_D, 8)
        pltpu.sync_copy(dst2_hbm.at[pl.ds(r, MAC_D)], idx_d)
        for j in range(MAC_D):
            pltpu.sync_copy(ones_v, acc.at[idx_d.at[j]], add=True)
        return 0
    lax.fori_loop(0, NMAC_D, body, 0)

    plsc.subcore_barrier()
    r0 = pl.multiple_of(s * (N_ACC1 // NS), 8)
    ro = pl.multiple_of(c * N_ACC1 + s * (N_ACC1 // NS), 8)
    pltpu.sync_copy(acc.at[pl.ds(r0, N_ACC1 // NS)], zbuf)
    pltpu.sync_copy(zbuf, out_hbm.at[pl.ds(ro, N_ACC1 // NS)])


_sc_params = pltpu.CompilerParams(use_tc_tiling_on_sc=False)

_deg_kernel = pl.kernel(
    _deg_body,
    out_type=jax.ShapeDtypeStruct((NC * N_ACC1,), jnp.float32),
    mesh=_mesh,
    compiler_params=_sc_params,
    scratch_types=[
        pltpu.VMEM((MAC_D, 128), jnp.int32),
        pltpu.VMEM((128,), jnp.float32),
        pltpu.VMEM((N_ACC1 // NS,), jnp.float32),
        pltpu.VMEM_SHARED((N_ACC1,), jnp.float32),
    ],
)


def _scatter_body(zs_a, zs_b, src1_hbm, dst2_hbm, out_hbm,
                  idx_s, idx_d, rows, acc, sems):
    c = lax.axis_index("c")
    s = lax.axis_index("s")

    st = rows.at[0]

    def init(i, _):
        st[i, :] = jnp.zeros((16,), jnp.float32)
        return 0
    lax.fori_loop(0, MACE, init, 0)
    for j in range(RPT // MACE):
        pltpu.sync_copy(
            st, acc.at[pl.ds(pl.multiple_of(s * RPT + j * MACE, 8), MACE)])
    pltpu.sync_copy(
        st.at[pl.ds(0, RPT % MACE)],
        acc.at[pl.ds(pl.multiple_of(s * RPT + MACE * (RPT // MACE), 8),
                     RPT % MACE)])
    plsc.subcore_barrier()

    def run(table):
        def fire(i):
            b = lax.rem(i, NBUF)
            e0 = pl.multiple_of(s * EPT + i * MACE, 8)
            r0 = pl.multiple_of(s * (EPT // 128) + i * MAC, 4)
            pltpu.sync_copy(src1_hbm.at[pl.ds(e0, MACE)], idx_s.at[b])
            pltpu.sync_copy(dst2_hbm.at[pl.ds(r0, MAC)], idx_d.at[b])
            pltpu.async_copy(table.at[idx_s.at[b]], rows.at[b], sems.at[b])

        for b in range(NBUF - 1):
            fire(b)

        def body(i, _):
            b = lax.rem(i, NBUF)
            pltpu.make_async_copy(table.at[idx_s.at[b]], rows.at[b],
                                  sems.at[b]).wait()

            @pl.when(i + NBUF - 1 < NMAC)
            def _():
                fire(i + NBUF - 1)

            for j in range(MAC):
                pltpu.sync_copy(rows.at[b, pl.ds(j * 128, 128)],
                                acc.at[idx_d.at[b, j]], add=True)
            return 0
        lax.fori_loop(0, NMAC, body, 0)

    @pl.when(c == 0)
    def _():
        run(zs_a)

    @pl.when(c == 1)
    def _():
        run(zs_b)

    plsc.subcore_barrier()
    for j in range(RPT // MACE):
        r0 = pl.multiple_of(s * RPT + j * MACE, 8)
        pltpu.sync_copy(acc.at[pl.ds(r0, MACE)], st)
        pltpu.sync_copy(st, out_hbm.at[c, pl.ds(r0, MACE)])
    rt = pl.multiple_of(s * RPT + MACE * (RPT // MACE), 8)
    pltpu.sync_copy(acc.at[pl.ds(rt, RPT % MACE)],
                    st.at[pl.ds(0, RPT % MACE)])
    pltpu.sync_copy(st.at[pl.ds(0, RPT % MACE)],
                    out_hbm.at[c, pl.ds(rt, RPT % MACE)])


_scatter_kernel = pl.kernel(
    _scatter_body,
    out_type=jax.ShapeDtypeStruct((NC, N_ACC, HH), jnp.float32),
    mesh=_mesh,
    compiler_params=_sc_params,
    scratch_types=[
        pltpu.VMEM((NBUF, MACE), jnp.int32),
        pltpu.VMEM((NBUF, MAC, 128), jnp.int32),
        pltpu.VMEM((NBUF, MACE, HH), jnp.float32),
        pltpu.VMEM_SHARED((N_ACC, HH), jnp.float32),
        pltpu.SemaphoreType.DMA((NBUF,)),
    ],
)


BN = 1000
NBLK = N // BN


def _k0_body(p0, p1, x, w0, dinv_o, z_o, zsa_o, zsb_o):
    deg = p0[...] + p1[...] + 1.0
    dinv = lax.rsqrt(deg)
    z = jnp.dot(x[...], w0[...], preferred_element_type=jnp.float32)
    zs = z * dinv
    dinv_o[...] = dinv
    z_o[...] = z
    zsa_o[...] = zs[:, :HH]
    zsb_o[...] = zs[:, HH:]


def _k0(p0, p1, xpad, w0pad):
    return pl.pallas_call(
        _k0_body,
        grid=(NBLK,),
        in_specs=[
            pl.BlockSpec((BN, 1), lambda i: (i, 0)),
            pl.BlockSpec((BN, 1), lambda i: (i, 0)),
            pl.BlockSpec((BN, 8), lambda i: (i, 0)),
            pl.BlockSpec((8, H), lambda i: (0, 0)),
        ],
        out_specs=[
            pl.BlockSpec((BN, 1), lambda i: (i, 0)),
            pl.BlockSpec((BN, H), lambda i: (i, 0)),
            pl.BlockSpec((BN, HH), lambda i: (i, 0)),
            pl.BlockSpec((BN, HH), lambda i: (i, 0)),
        ],
        out_shape=[
            jax.ShapeDtypeStruct((N, 1), jnp.float32),
            jax.ShapeDtypeStruct((N, H), jnp.float32),
            jax.ShapeDtypeStruct((N, HH), jnp.float32),
            jax.ShapeDtypeStruct((N, HH), jnp.float32),
        ],
    )(p0, p1, xpad, w0pad)


def _kmid_body(sa, sb, z_prev, dinv, b, w, z_o, zsa_o, zsb_o):
    di = dinv[...]
    agg = jnp.concatenate([sa[...], sb[...]], axis=1) * di \
        + z_prev[...] * (di * di)
    h = jnp.maximum(agg + b[...], 0.0)
    z = jnp.dot(h, w[...], preferred_element_type=jnp.float32)
    zs = z * di
    z_o[...] = z
    zsa_o[...] = zs[:, :HH]
    zsb_o[...] = zs[:, HH:]


def _kmid(sa, sb, z_prev, dinv, b, w):
    return pl.pallas_call(
        _kmid_body,
        grid=(NBLK,),
        in_specs=[
            pl.BlockSpec((BN, HH), lambda i: (i, 0)),
            pl.BlockSpec((BN, HH), lambda i: (i, 0)),
            pl.BlockSpec((BN, H), lambda i: (i, 0)),
            pl.BlockSpec((BN, 1), lambda i: (i, 0)),
            pl.BlockSpec((1, H), lambda i: (0, 0)),
            pl.BlockSpec((H, H), lambda i: (0, 0)),
        ],
        out_specs=[
            pl.BlockSpec((BN, H), lambda i: (i, 0)),
            pl.BlockSpec((BN, HH), lambda i: (i, 0)),
            pl.BlockSpec((BN, HH), lambda i: (i, 0)),
        ],
        out_shape=[
            jax.ShapeDtypeStruct((N, H), jnp.float32),
            jax.ShapeDtypeStruct((N, HH), jnp.float32),
            jax.ShapeDtypeStruct((N, HH), jnp.float32),
        ],
    )(sa, sb, z_prev, dinv, b, w)


def _kfin_body(sa, sb, z_prev, dinv, b, batch, fw1, fb1, fw2, fb2,
               out_o, sums, cnt):
    i = pl.program_id(0)

    @pl.when(i == 0)
    def _():
        sums[...] = jnp.zeros_like(sums)
        cnt[...] = jnp.zeros_like(cnt)

    di = dinv[...]
    agg = jnp.concatenate([sa[...], sb[...]], axis=1) * di \
        + z_prev[...] * (di * di)
    h = jnp.maximum(agg + b[...], 0.0)

    bb = batch[...].reshape(1, BN)
    gid = lax.broadcasted_iota(jnp.int32, (G, BN), 0)
    onehot = (gid == bb).astype(jnp.float32)
    sums[...] += jnp.dot(onehot, h, preferred_element_type=jnp.float32)
    cnt[...] += jnp.sum(onehot, axis=1, keepdims=True)

    @pl.when(i == NBLK - 1)
    def _():
        pooled = sums[...] / jnp.maximum(cnt[...], 1.0)
        r = jnp.maximum(
            jnp.dot(pooled, fw1[...], preferred_element_type=jnp.float32)
            + fb1[...], 0.0)
        out_o[...] = (
            jnp.dot(r, fw2[...], preferred_element_type=jnp.float32)
            + fb2[...])


def _kfin(sa, sb, z_prev, dinv, b, batch3, fw1, fb1, fw2, fb2):
    return pl.pallas_call(
        _kfin_body,
        grid=(NBLK,),
        in_specs=[
            pl.BlockSpec((BN, HH), lambda i: (i, 0)),
            pl.BlockSpec((BN, HH), lambda i: (i, 0)),
            pl.BlockSpec((BN, H), lambda i: (i, 0)),
            pl.BlockSpec((BN, 1), lambda i: (i, 0)),
            pl.BlockSpec((1, H), lambda i: (0, 0)),
            pl.BlockSpec((1, 1, BN), lambda i: (i, 0, 0)),
            pl.BlockSpec((H, DENSE), lambda i: (0, 0)),
            pl.BlockSpec((1, DENSE), lambda i: (0, 0)),
            pl.BlockSpec((DENSE, 1), lambda i: (0, 0)),
            pl.BlockSpec((1, 1), lambda i: (0, 0)),
        ],
        out_specs=pl.BlockSpec((G, 1), lambda i: (0, 0)),
        out_shape=jax.ShapeDtypeStruct((G, 1), jnp.float32),
        scratch_shapes=[
            pltpu.VMEM((G, H), jnp.float32),
            pltpu.VMEM((G, 1), jnp.float32),
        ],
    )(sa, sb, z_prev, dinv, b, batch3, fw1, fb1, fw2, fb2)


@jax.jit
def kernel(x, edge_index, batch, W0, b0, W1, b1, W2, b2, fW1, fb1, fW2, fb2):
    pad = E_PAD - E
    src1 = jnp.concatenate([edge_index[0], jnp.zeros((pad,), jnp.int32)])
    dstp = jnp.concatenate([edge_index[1], jnp.full((pad,), N, jnp.int32)])
    dst2 = dstp.reshape(E_PAD // 128, 128)

    xpad = jnp.pad(x, ((0, 0), (0, 8 - x.shape[1])))
    w0pad = jnp.pad(W0, ((0, 8 - W0.shape[0]), (0, 0)))

    degp = _deg_kernel(dst2)
    p0 = degp[:N].reshape(N, 1)
    p1 = degp[N_ACC1:N_ACC1 + N].reshape(N, 1)

    dinv, z0, zs0a, zs0b = _k0(p0, p1, xpad, w0pad)
    s0 = _scatter_kernel(zs0a, zs0b, src1, dst2)
    z1, zs1a, zs1b = _kmid(s0[0, :N], s0[1, :N], z0, dinv,
                           b0.reshape(1, H), W1)
    s1 = _scatter_kernel(zs1a, zs1b, src1, dst2)
    z2, zs2a, zs2b = _kmid(s1[0, :N], s1[1, :N], z1, dinv,
                           b1.reshape(1, H), W2)
    s2 = _scatter_kernel(zs2a, zs2b, src1, dst2)

    batch3 = batch.reshape(NBLK, 1, BN)
    return _kfin(s2[0, :N], s2[1, :N], z2, dinv, b2.reshape(1, H), batch3,
                 fW1, fb1.reshape(1, DENSE), fW2, fb2.reshape(1, 1))


# speedup vs baseline: 22.9214x; 1.1030x over previous
import functools

import jax
import jax.numpy as jnp
from jax import lax
from jax.experimental import pallas as pl
from jax.experimental.pallas import tpu as pltpu
from jax.experimental.pallas import tpu_sc as plsc

N = 100000
E = 1600000
H = 32
HH = 16
G = 64
DENSE = 64

NC = 2
NS = 16

EPT = 100352
E_PAD = EPT * NS
MAC = 4
MACE = MAC * 128
NMAC = EPT // MACE
NBUF = 3
EPT_D = E_PAD // 2 // NS
MAC_D = 8
NMAC_D = EPT_D // (MAC_D * 128)

N_ACC = 100352
N_ACC1 = 100096
RPT = N_ACC // NS

_mesh = plsc.VectorSubcoreMesh(core_axis_name="c", subcore_axis_name="s")


def _deg_body(dst2_hbm, out_hbm, idx_d, ones_v, zbuf, acc):
    c = lax.axis_index("c")
    s = lax.axis_index("s")

    def init(i, _):
        zbuf[pl.ds(i * 16, 16)] = jnp.zeros((16,), jnp.float32)
        return 0
    lax.fori_loop(0, N_ACC1 // NS // 16, init, 0)
    for j in range(8):
        ones_v[pl.ds(j * 16, 16)] = jnp.ones((16,), jnp.float32)
    pltpu.sync_copy(
        zbuf, acc.at[pl.ds(pl.multiple_of(s * (N_ACC1 // NS), 8),
                           N_ACC1 // NS)])
    plsc.subcore_barrier()

    base_row = (c * (E_PAD // 2) + s * EPT_D) // 128

    def body(i, _):
        r = pl.multiple_of(base_row + i * MAC_D, 8)
        pltpu.sync_copy(dst2_hbm.at[pl.ds(r, MAC_D)], idx_d)
        for j in range(MAC_D):
            pltpu.sync_copy(ones_v, acc.at[idx_d.at[j]], add=True)
        return 0
    lax.fori_loop(0, NMAC_D, body, 0)

    plsc.subcore_barrier()
    r0 = pl.multiple_of(s * (N_ACC1 // NS), 8)
    ro = pl.multiple_of(c * N_ACC1 + s * (N_ACC1 // NS), 8)
    pltpu.sync_copy(acc.at[pl.ds(r0, N_ACC1 // NS)], zbuf)
    pltpu.sync_copy(zbuf, out_hbm.at[pl.ds(ro, N_ACC1 // NS)])


_sc_params = pltpu.CompilerParams(use_tc_tiling_on_sc=False)

_deg_kernel = pl.kernel(
    _deg_body,
    out_type=jax.ShapeDtypeStruct((NC * N_ACC1,), jnp.float32),
    mesh=_mesh,
    compiler_params=_sc_params,
    scratch_types=[
        pltpu.VMEM((MAC_D, 128), jnp.int32),
        pltpu.VMEM((128,), jnp.float32),
        pltpu.VMEM((N_ACC1 // NS,), jnp.float32),
        pltpu.VMEM_SHARED((N_ACC1,), jnp.float32),
    ],
)


def _scatter_body(zs_a, zs_b, src1_hbm, dst2_hbm, out_hbm,
                  idx_s, idx_d, rows, acc, sems, ssems):
    c = lax.axis_index("c")
    s = lax.axis_index("s")

    st = rows.at[0]

    def init(i, _):
        st[i, :] = jnp.zeros((16,), jnp.float32)
        return 0
    lax.fori_loop(0, MACE, init, 0)
    for j in range(RPT // MACE):
        pltpu.sync_copy(
            st, acc.at[pl.ds(pl.multiple_of(s * RPT + j * MACE, 8), MACE)])
    pltpu.sync_copy(
        st.at[pl.ds(0, RPT % MACE)],
        acc.at[pl.ds(pl.multiple_of(s * RPT + MACE * (RPT // MACE), 8),
                     RPT % MACE)])
    plsc.subcore_barrier()

    def run(table):
        def fire(i):
            b = lax.rem(i, NBUF)
            e0 = pl.multiple_of(s * EPT + i * MACE, 8)
            r0 = pl.multiple_of(s * (EPT // 128) + i * MAC, 4)
            pltpu.sync_copy(src1_hbm.at[pl.ds(e0, MACE)], idx_s.at[b])
            pltpu.sync_copy(dst2_hbm.at[pl.ds(r0, MAC)], idx_d.at[b])
            pltpu.async_copy(table.at[idx_s.at[b]], rows.at[b], sems.at[b])

        def drain_scatter(b):
            for j in range(MAC):
                pltpu.make_async_copy(rows.at[b, pl.ds(j * 128, 128)],
                                      acc.at[idx_d.at[b, j]],
                                      ssems.at[b]).wait()

        for b in range(NBUF - 1):
            fire(b)

        def body(i, _):
            b = lax.rem(i, NBUF)
            pltpu.make_async_copy(table.at[idx_s.at[b]], rows.at[b],
                                  sems.at[b]).wait()

            for j in range(MAC):
                pltpu.async_copy(rows.at[b, pl.ds(j * 128, 128)],
                                 acc.at[idx_d.at[b, j]],
                                 ssems.at[b], add=True)

            nxt = i + NBUF - 1

            @pl.when(nxt < NMAC)
            def _():
                b2 = lax.rem(nxt, NBUF)

                @pl.when(i >= 1)
                def _():
                    drain_scatter(b2)
                fire(nxt)
            return 0
        lax.fori_loop(0, NMAC, body, 0)
        for k in range(NBUF):
            drain_scatter(k)

    @pl.when(c == 0)
    def _():
        run(zs_a)

    @pl.when(c == 1)
    def _():
        run(zs_b)

    plsc.subcore_barrier()
    for j in range(RPT // MACE):
        r0 = pl.multiple_of(s * RPT + j * MACE, 8)
        pltpu.sync_copy(acc.at[pl.ds(r0, MACE)], st)
        pltpu.sync_copy(st, out_hbm.at[c, pl.ds(r0, MACE)])
    rt = pl.multiple_of(s * RPT + MACE * (RPT // MACE), 8)
    pltpu.sync_copy(acc.at[pl.ds(rt, RPT % MACE)],
                    st.at[pl.ds(0, RPT % MACE)])
    pltpu.sync_copy(st.at[pl.ds(0, RPT % MACE)],
                    out_hbm.at[c, pl.ds(rt, RPT % MACE)])


_scatter_kernel = pl.kernel(
    _scatter_body,
    out_type=jax.ShapeDtypeStruct((NC, N_ACC, HH), jnp.float32),
    mesh=_mesh,
    compiler_params=_sc_params,
    scratch_types=[
        pltpu.VMEM((NBUF, MACE), jnp.int32),
        pltpu.VMEM((NBUF, MAC, 128), jnp.int32),
        pltpu.VMEM((NBUF, MACE, HH), jnp.float32),
        pltpu.VMEM_SHARED((N_ACC, HH), jnp.float32),
        pltpu.SemaphoreType.DMA((NBUF,)),
        pltpu.SemaphoreType.DMA((NBUF,)),
    ],
)


BN = 1000
NBLK = N // BN


def _k0_body(p0, p1, x, w0, dinv_o, z_o, zsa_o, zsb_o):
    deg = p0[...] + p1[...] + 1.0
    dinv = lax.rsqrt(deg)
    z = jnp.dot(x[...], w0[...], preferred_element_type=jnp.float32)
    zs = z * dinv
    dinv_o[...] = dinv
    z_o[...] = z
    zsa_o[...] = zs[:, :HH]
    zsb_o[...] = zs[:, HH:]


def _k0(p0, p1, xpad, w0pad):
    return pl.pallas_call(
        _k0_body,
        grid=(NBLK,),
        in_specs=[
            pl.BlockSpec((BN, 1), lambda i: (i, 0)),
            pl.BlockSpec((BN, 1), lambda i: (i, 0)),
            pl.BlockSpec((BN, 8), lambda i: (i, 0)),
            pl.BlockSpec((8, H), lambda i: (0, 0)),
        ],
        out_specs=[
            pl.BlockSpec((BN, 1), lambda i: (i, 0)),
            pl.BlockSpec((BN, H), lambda i: (i, 0)),
            pl.BlockSpec((BN, HH), lambda i: (i, 0)),
            pl.BlockSpec((BN, HH), lambda i: (i, 0)),
        ],
        out_shape=[
            jax.ShapeDtypeStruct((N, 1), jnp.float32),
            jax.ShapeDtypeStruct((N, H), jnp.float32),
            jax.ShapeDtypeStruct((N, HH), jnp.float32),
            jax.ShapeDtypeStruct((N, HH), jnp.float32),
        ],
    )(p0, p1, xpad, w0pad)


def _kmid_body(sa, sb, z_prev, dinv, b, w, z_o, zsa_o, zsb_o):
    di = dinv[...]
    agg = jnp.concatenate([sa[...], sb[...]], axis=1) * di \
        + z_prev[...] * (di * di)
    h = jnp.maximum(agg + b[...], 0.0)
    z = jnp.dot(h, w[...], preferred_element_type=jnp.float32)
    zs = z * di
    z_o[...] = z
    zsa_o[...] = zs[:, :HH]
    zsb_o[...] = zs[:, HH:]


def _kmid(sa, sb, z_prev, dinv, b, w):
    return pl.pallas_call(
        _kmid_body,
        grid=(NBLK,),
        in_specs=[
            pl.BlockSpec((BN, HH), lambda i: (i, 0)),
            pl.BlockSpec((BN, HH), lambda i: (i, 0)),
            pl.BlockSpec((BN, H), lambda i: (i, 0)),
            pl.BlockSpec((BN, 1), lambda i: (i, 0)),
            pl.BlockSpec((1, H), lambda i: (0, 0)),
            pl.BlockSpec((H, H), lambda i: (0, 0)),
        ],
        out_specs=[
            pl.BlockSpec((BN, H), lambda i: (i, 0)),
            pl.BlockSpec((BN, HH), lambda i: (i, 0)),
            pl.BlockSpec((BN, HH), lambda i: (i, 0)),
        ],
        out_shape=[
            jax.ShapeDtypeStruct((N, H), jnp.float32),
            jax.ShapeDtypeStruct((N, HH), jnp.float32),
            jax.ShapeDtypeStruct((N, HH), jnp.float32),
        ],
    )(sa, sb, z_prev, dinv, b, w)


def _kfin_body(sa, sb, z_prev, dinv, b, batch, fw1, fb1, fw2, fb2,
               out_o, sums, cnt):
    i = pl.program_id(0)

    @pl.when(i == 0)
    def _():
        sums[...] = jnp.zeros_like(sums)
        cnt[...] = jnp.zeros_like(cnt)

    di = dinv[...]
    agg = jnp.concatenate([sa[...], sb[...]], axis=1) * di \
        + z_prev[...] * (di * di)
    h = jnp.maximum(agg + b[...], 0.0)

    bb = batch[...].reshape(1, BN)
    gid = lax.broadcasted_iota(jnp.int32, (G, BN), 0)
    onehot = (gid == bb).astype(jnp.float32)
    sums[...] += jnp.dot(onehot, h, preferred_element_type=jnp.float32)
    cnt[...] += jnp.sum(onehot, axis=1, keepdims=True)

    @pl.when(i == NBLK - 1)
    def _():
        pooled = sums[...] / jnp.maximum(cnt[...], 1.0)
        r = jnp.maximum(
            jnp.dot(pooled, fw1[...], preferred_element_type=jnp.float32)
            + fb1[...], 0.0)
        out_o[...] = (
            jnp.dot(r, fw2[...], preferred_element_type=jnp.float32)
            + fb2[...])


def _kfin(sa, sb, z_prev, dinv, b, batch3, fw1, fb1, fw2, fb2):
    return pl.pallas_call(
        _kfin_body,
        grid=(NBLK,),
        in_specs=[
            pl.BlockSpec((BN, HH), lambda i: (i, 0)),
            pl.BlockSpec((BN, HH), lambda i: (i, 0)),
            pl.BlockSpec((BN, H), lambda i: (i, 0)),
            pl.BlockSpec((BN, 1), lambda i: (i, 0)),
            pl.BlockSpec((1, H), lambda i: (0, 0)),
            pl.BlockSpec((1, 1, BN), lambda i: (i, 0, 0)),
            pl.BlockSpec((H, DENSE), lambda i: (0, 0)),
            pl.BlockSpec((1, DENSE), lambda i: (0, 0)),
            pl.BlockSpec((DENSE, 1), lambda i: (0, 0)),
            pl.BlockSpec((1, 1), lambda i: (0, 0)),
        ],
        out_specs=pl.BlockSpec((G, 1), lambda i: (0, 0)),
        out_shape=jax.ShapeDtypeStruct((G, 1), jnp.float32),
        scratch_shapes=[
            pltpu.VMEM((G, H), jnp.float32),
            pltpu.VMEM((G, 1), jnp.float32),
        ],
    )(sa, sb, z_prev, dinv, b, batch3, fw1, fb1, fw2, fb2)


@jax.jit
def kernel(x, edge_index, batch, W0, b0, W1, b1, W2, b2, fW1, fb1, fW2, fb2):
    pad = E_PAD - E
    src1 = jnp.concatenate([edge_index[0], jnp.zeros((pad,), jnp.int32)])
    dstp = jnp.concatenate([edge_index[1], jnp.full((pad,), N, jnp.int32)])
    dst2 = dstp.reshape(E_PAD // 128, 128)

    xpad = jnp.pad(x, ((0, 0), (0, 8 - x.shape[1])))
    w0pad = jnp.pad(W0, ((0, 8 - W0.shape[0]), (0, 0)))

    degp = _deg_kernel(dst2)
    p0 = degp[:N].reshape(N, 1)
    p1 = degp[N_ACC1:N_ACC1 + N].reshape(N, 1)

    dinv, z0, zs0a, zs0b = _k0(p0, p1, xpad, w0pad)
    s0 = _scatter_kernel(zs0a, zs0b, src1, dst2)
    z1, zs1a, zs1b = _kmid(s0[0, :N], s0[1, :N], z0, dinv,
                           b0.reshape(1, H), W1)
    s1 = _scatter_kernel(zs1a, zs1b, src1, dst2)
    z2, zs2a, zs2b = _kmid(s1[0, :N], s1[1, :N], z1, dinv,
                           b1.reshape(1, H), W2)
    s2 = _scatter_kernel(zs2a, zs2b, src1, dst2)

    batch3 = batch.reshape(NBLK, 1, BN)
    return _kfin(s2[0, :N], s2[1, :N], z2, dinv, b2.reshape(1, H), batch3,
                 fW1, fb1.reshape(1, DENSE), fW2, fb2.reshape(1, 1))


# speedup vs baseline: 35.2334x; 1.5371x over previous
import jax
import jax.numpy as jnp
from jax import lax
from jax.experimental import pallas as pl
from jax.experimental.pallas import tpu as pltpu
from jax.experimental.pallas import tpu_sc as plsc

N = 100000
E = 1600000
H = 32
HH = 16
G = 64
DENSE = 64

NP = 100352
BN = 1024
NBLK = NP // BN
PR = NP * H // 128
BR = BN * H // 128

NC = 2
NS = 16

EPT = 100352
E_PAD = EPT * NS
MAC = 4
MACE = MAC * 128
NMAC = EPT // MACE
NBUF = 3
EPT_D = E_PAD // 2 // NS
MAC_D = 8
NMAC_D = EPT_D // (MAC_D * 128)

N_ACC = NP
RPT = N_ACC // NS

_mesh = plsc.VectorSubcoreMesh(core_axis_name="c", subcore_axis_name="s")
_sc_params = pltpu.CompilerParams(use_tc_tiling_on_sc=False)


def _deg_body(dst2_hbm, out_hbm, idx_d, ones_v, zbuf, acc):
    c = lax.axis_index("c")
    s = lax.axis_index("s")

    def init(i, _):
        zbuf[pl.ds(i * 16, 16)] = jnp.zeros((16,), jnp.float32)
        return 0
    lax.fori_loop(0, RPT // 16, init, 0)
    for j in range(8):
        ones_v[pl.ds(j * 16, 16)] = jnp.ones((16,), jnp.float32)
    pltpu.sync_copy(zbuf, acc.at[pl.ds(pl.multiple_of(s * RPT, 8), RPT)])
    plsc.subcore_barrier()

    base_row = (c * (E_PAD // 2) + s * EPT_D) // 128

    def body(i, _):
        r = pl.multiple_of(base_row + i * MAC_D, 8)
        pltpu.sync_copy(dst2_hbm.at[pl.ds(r, MAC_D)], idx_d)
        for j in range(MAC_D):
            pltpu.sync_copy(ones_v, acc.at[idx_d.at[j]], add=True)
        return 0
    lax.fori_loop(0, NMAC_D, body, 0)

    plsc.subcore_barrier()
    r0 = pl.multiple_of(s * RPT, 8)
    ro = pl.multiple_of(c * NP + s * RPT, 8)
    pltpu.sync_copy(acc.at[pl.ds(r0, RPT)], zbuf)
    pltpu.sync_copy(zbuf, out_hbm.at[pl.ds(ro, RPT)])


_deg_kernel = pl.kernel(
    _deg_body,
    out_type=jax.ShapeDtypeStruct((NC * NP,), jnp.float32),
    mesh=_mesh,
    compiler_params=_sc_params,
    scratch_types=[
        pltpu.VMEM((MAC_D, 128), jnp.int32),
        pltpu.VMEM((128,), jnp.float32),
        pltpu.VMEM((RPT,), jnp.float32),
        pltpu.VMEM_SHARED((NP,), jnp.float32),
    ],
)


def _scatter_body(zs2, srca_hbm, srcb_hbm, dst2_hbm, out_hbm,
                  idx_s, idx_d, rows, acc, sems, ssems):
    c = lax.axis_index("c")
    s = lax.axis_index("s")

    st = rows.at[0]

    def init(i, _):
        st[i, :] = jnp.zeros((16,), jnp.float32)
        return 0
    lax.fori_loop(0, MACE, init, 0)
    for j in range(RPT // MACE):
        pltpu.sync_copy(
            st, acc.at[pl.ds(pl.multiple_of(s * RPT + j * MACE, 8), MACE)])
    pltpu.sync_copy(
        st.at[pl.ds(0, RPT % MACE)],
        acc.at[pl.ds(pl.multiple_of(s * RPT + MACE * (RPT // MACE), 8),
                     RPT % MACE)])
    plsc.subcore_barrier()

    def run(src1_hbm):
        def fire(i):
            b = lax.rem(i, NBUF)
            e0 = pl.multiple_of(s * EPT + i * MACE, 8)
            r0 = pl.multiple_of(s * (EPT // 128) + i * MAC, 4)
            pltpu.sync_copy(src1_hbm.at[pl.ds(e0, MACE)], idx_s.at[b])
            pltpu.sync_copy(dst2_hbm.at[pl.ds(r0, MAC)], idx_d.at[b])
            pltpu.async_copy(zs2.at[idx_s.at[b]], rows.at[b], sems.at[b])

        def drain_scatter(b):
            for j in range(MAC):
                pltpu.make_async_copy(rows.at[b, pl.ds(j * 128, 128)],
                                      acc.at[idx_d.at[b, j]],
                                      ssems.at[b]).wait()

        for b in range(NBUF - 1):
            fire(b)

        def body(i, _):
            b = lax.rem(i, NBUF)
            pltpu.make_async_copy(zs2.at[idx_s.at[b]], rows.at[b],
                                  sems.at[b]).wait()

            for j in range(MAC):
                pltpu.async_copy(rows.at[b, pl.ds(j * 128, 128)],
                                 acc.at[idx_d.at[b, j]],
                                 ssems.at[b], add=True)

            nxt = i + NBUF - 1

            @pl.when(nxt < NMAC)
            def _():
                b2 = lax.rem(nxt, NBUF)

                @pl.when(i >= 1)
                def _():
                    drain_scatter(b2)
                fire(nxt)
            return 0
        lax.fori_loop(0, NMAC, body, 0)
        for k in range(NBUF):
            drain_scatter(k)

    @pl.when(c == 0)
    def _():
        run(srca_hbm)

    @pl.when(c == 1)
    def _():
        run(srcb_hbm)

    plsc.subcore_barrier()
    for j in range(RPT // MACE):
        r0 = pl.multiple_of(s * RPT + j * MACE, 8)
        pltpu.sync_copy(acc.at[pl.ds(r0, MACE)], st)
        pltpu.sync_copy(st, out_hbm.at[pl.ds(r0, MACE), c])
    rt = pl.multiple_of(s * RPT + MACE * (RPT // MACE), 8)
    pltpu.sync_copy(acc.at[pl.ds(rt, RPT % MACE)],
                    st.at[pl.ds(0, RPT % MACE)])
    pltpu.sync_copy(st.at[pl.ds(0, RPT % MACE)],
                    out_hbm.at[pl.ds(rt, RPT % MACE), c])


_scatter_kernel = pl.kernel(
    _scatter_body,
    out_type=jax.ShapeDtypeStruct((NP, NC, HH), jnp.float32),
    mesh=_mesh,
    compiler_params=_sc_params,
    scratch_types=[
        pltpu.VMEM((NBUF, MACE), jnp.int32),
        pltpu.VMEM((NBUF, MAC, 128), jnp.int32),
        pltpu.VMEM((NBUF, MACE, HH), jnp.float32),
        pltpu.VMEM_SHARED((N_ACC, HH), jnp.float32),
        pltpu.SemaphoreType.DMA((NBUF,)),
        pltpu.SemaphoreType.DMA((NBUF,)),
    ],
)


def _k0_body(d32, x, w0, z_o, zs_o):
    z = jnp.dot(x[...], w0[...], preferred_element_type=jnp.float32)
    zs = z * lax.rsqrt(d32[...])
    z_o[...] = z
    zs_o[...] = zs


def _k0(d32, xp32, bdw0):
    return pl.pallas_call(
        _k0_body,
        grid=(NBLK,),
        in_specs=[
            pl.BlockSpec((BR, 128), lambda i: (i, 0)),
            pl.BlockSpec((BR, 128), lambda i: (i, 0)),
            pl.BlockSpec((128, 128), lambda i: (0, 0)),
        ],
        out_specs=[
            pl.BlockSpec((BR, 128), lambda i: (i, 0)),
            pl.BlockSpec((BR, 128), lambda i: (i, 0)),
        ],
        out_shape=[
            jax.ShapeDtypeStruct((PR, 128), jnp.float32),
            jax.ShapeDtypeStruct((PR, 128), jnp.float32),
        ],
    )(d32, xp32, bdw0)


def _agg_h(s_pk, z_prev, d32, b4):
    di = lax.rsqrt(d32[...])
    agg = s_pk[...] * di + z_prev[...] * (di * di)
    return jnp.maximum(agg + b4[...], 0.0), di


def _kmid_body(s_pk, z_prev, d32, b4, bdw, z_o, zs_o):
    h, di = _agg_h(s_pk, z_prev, d32, b4)
    z = jnp.dot(h, bdw[...], preferred_element_type=jnp.float32)
    z_o[...] = z
    zs_o[...] = z * di


def _kmid(s_pk, z_pk, d32, b4, bdw):
    return pl.pallas_call(
        _kmid_body,
        grid=(NBLK,),
        in_specs=[
            pl.BlockSpec((BR, 128), lambda i: (i, 0)),
            pl.BlockSpec((BR, 128), lambda i: (i, 0)),
            pl.BlockSpec((BR, 128), lambda i: (i, 0)),
            pl.BlockSpec((1, 128), lambda i: (0, 0)),
            pl.BlockSpec((128, 128), lambda i: (0, 0)),
        ],
        out_specs=[
            pl.BlockSpec((BR, 128), lambda i: (i, 0)),
            pl.BlockSpec((BR, 128), lambda i: (i, 0)),
        ],
        out_shape=[
            jax.ShapeDtypeStruct((PR, 128), jnp.float32),
            jax.ShapeDtypeStruct((PR, 128), jnp.float32),
        ],
    )(s_pk, z_pk, d32, b4, bdw)


def _kfin_body(s_pk, z_prev, d32, b4, batch4, fw1, fb1, fw2, fb2,
               out_o, sums, cnt):
    i = pl.program_id(0)

    @pl.when(i == 0)
    def _():
        sums[...] = jnp.zeros_like(sums)
        cnt[...] = jnp.zeros_like(cnt)

    h, _ = _agg_h(s_pk, z_prev, d32, b4)

    bq = batch4[...].reshape(4, BR)
    gid = lax.broadcasted_iota(jnp.int32, (G, BR), 0)
    for q in range(4):
        onehot = (gid == bq[q:q + 1, :]).astype(jnp.float32)
        sums[...] += jnp.dot(onehot, h[:, 32 * q:32 * (q + 1)],
                             preferred_element_type=jnp.float32)
        cnt[...] += jnp.sum(onehot, axis=1, keepdims=True)

    @pl.when(i == NBLK - 1)
    def _():
        pooled = sums[...] / jnp.maximum(cnt[...], 1.0)
        r = jnp.maximum(
            jnp.dot(pooled, fw1[...], preferred_element_type=jnp.float32)
            + fb1[...], 0.0)
        out_o[...] = (
            jnp.dot(r, fw2[...], preferred_element_type=jnp.float32)
            + fb2[...])


def _kfin(s_pk, z_pk, d32, b4, batch4, fw1, fb1, fw2, fb2):
    return pl.pallas_call(
        _kfin_body,
        grid=(NBLK,),
        in_specs=[
            pl.BlockSpec((BR, 128), lambda i: (i, 0)),
            pl.BlockSpec((BR, 128), lambda i: (i, 0)),
            pl.BlockSpec((BR, 128), lambda i: (i, 0)),
            pl.BlockSpec((1, 128), lambda i: (0, 0)),
            pl.BlockSpec((1, 4, BR), lambda i: (i, 0, 0)),
            pl.BlockSpec((H, DENSE), lambda i: (0, 0)),
            pl.BlockSpec((1, DENSE), lambda i: (0, 0)),
            pl.BlockSpec((DENSE, 1), lambda i: (0, 0)),
            pl.BlockSpec((1, 1), lambda i: (0, 0)),
        ],
        out_specs=pl.BlockSpec((G, 1), lambda i: (0, 0)),
        out_shape=jax.ShapeDtypeStruct((G, 1), jnp.float32),
        scratch_shapes=[
            pltpu.VMEM((G, H), jnp.float32),
            pltpu.VMEM((G, 1), jnp.float32),
        ],
    )(s_pk, z_pk, d32, b4, batch4, fw1, fb1, fw2, fb2)


@jax.jit
def kernel(x, edge_index, batch, W0, b0, W1, b1, W2, b2, fW1, fb1, fW2, fb2):
    pad = E_PAD - E
    src1 = jnp.concatenate([edge_index[0], jnp.zeros((pad,), jnp.int32)])
    srca = src1 * 2
    srcb = srca + 1
    dstp = jnp.concatenate([edge_index[1], jnp.full((pad,), N, jnp.int32)])
    dst2 = dstp.reshape(E_PAD // 128, 128)

    xp32 = jnp.pad(x, ((0, NP - N), (0, H - x.shape[1]))).reshape(-1, 128)
    bdw0 = jnp.kron(jnp.eye(4, dtype=jnp.float32),
                    jnp.pad(W0, ((0, H - W0.shape[0]), (0, 0))))
    bdw1 = jnp.kron(jnp.eye(4, dtype=jnp.float32), W1)
    bdw2 = jnp.kron(jnp.eye(4, dtype=jnp.float32), W2)
    batch_p = jnp.concatenate([batch, jnp.full((NP - N,), G, jnp.int32)])
    batch4 = batch_p.reshape(PR, 4).T.reshape(4, NBLK, BR).transpose(1, 0, 2)

    degp = _deg_kernel(dst2)
    d32 = jnp.repeat(degp[:NP] + degp[NP:] + 1.0, H).reshape(PR, 128)

    z0, zs0 = _k0(d32, xp32, bdw0)

    def tab(zs):
        return zs.reshape(NC * NP, HH)

    def spk(s):
        return s.reshape(PR, 128)

    s0 = spk(_scatter_kernel(tab(zs0), srca, srcb, dst2))
    z1, zs1 = _kmid(s0, z0, d32, jnp.tile(b0, 4).reshape(1, 128), bdw1)
    s1 = spk(_scatter_kernel(tab(zs1), srca, srcb, dst2))
    z2, zs2 = _kmid(s1, z1, d32, jnp.tile(b1, 4).reshape(1, 128), bdw2)
    s2 = spk(_scatter_kernel(tab(zs2), srca, srcb, dst2))

    return _kfin(s2, z2, d32, jnp.tile(b2, 4).reshape(1, 128), batch4,
                 fW1, fb1.reshape(1, DENSE), fW2, fb2.reshape(1, 1))
